# baseline (speedup 1.0000x reference)
import functools

import jax
import jax.numpy as jnp
from jax import lax
from jax.experimental import pallas as pl
from jax.experimental.pallas import tpu as pltpu
from jax.experimental.pallas import tpu_sc as plsc

N_NODES = 100000
N_EDGES = 3200000
D_FEAT = 128
D_MEM = 128
D_EDGE = 16
D_TIME = 128
D_EMB = 128
N_HEADS = 2
B = 4096
K = 20
QD = D_EMB + D_TIME
KD = D_EMB + D_EDGE + D_TIME
DH = QD // N_HEADS

BB = 256
NB = B // BB
PB = BB * K + BB
NIDX = NB * PB
WIN = 128

PROJ_BB = 1000
PROJ_NB = N_NODES // PROJ_BB


def _project_body(nf, mem, Wp, bp, out):
    Wp_ = Wp[...]
    out[...] = (jnp.dot(nf[...], Wp_[:D_FEAT],
                        preferred_element_type=jnp.float32)
                + jnp.dot(mem[...], Wp_[D_FEAT:],
                          preferred_element_type=jnp.float32)
                + bp[...])


def _project_tables(node_features, memory_tbl, Wp, bp):
    return pl.pallas_call(
        _project_body,
        grid=(PROJ_NB,),
        in_specs=[
            pl.BlockSpec((PROJ_BB, D_FEAT), lambda i: (i, 0)),
            pl.BlockSpec((PROJ_BB, D_MEM), lambda i: (i, 0)),
            pl.BlockSpec(Wp.shape, lambda i: (0, 0)),
            pl.BlockSpec(bp.shape, lambda i: (0, 0)),
        ],
        out_specs=pl.BlockSpec((PROJ_BB, D_EMB), lambda i: (i, 0)),
        out_shape=jax.ShapeDtypeStruct((N_NODES, D_EMB), jnp.float32),
        compiler_params=pltpu.CompilerParams(
            dimension_semantics=("parallel",)),
    )(node_features, memory_tbl, Wp, bp)


def _sc_gather_emb(emb_tbl, nidx):
    mesh = plsc.VectorSubcoreMesh(core_axis_name="c", subcore_axis_name="s")

    @functools.partial(
        pl.kernel,
        out_type=jax.ShapeDtypeStruct((NIDX, D_EMB), jnp.float32),
        mesh=mesh,
    )
    def gather_kernel(emb_hbm, nidx_hbm, emb_out):
        def body(nidx_v, emb_o):
            pltpu.sync_copy(emb_hbm.at[nidx_v.at[0]], emb_o)

        pltpu.emit_pipeline(
            body,
            grid=(NIDX // WIN,),
            in_specs=[pl.BlockSpec((1, WIN), lambda i: (0, i))],
            out_specs=[pl.BlockSpec((WIN, D_EMB), lambda i: (i, 0))],
            core_axis_name=("c", "s"),
            dimension_semantics=(pltpu.PARALLEL,),
        )(nidx_hbm, emb_out)

    return gather_kernel(emb_tbl, nidx)


def _fast_cos(x):
    n = jnp.round(x * 0.15915494309189535)
    y = x - n * 6.28125 - n * 1.9353071795864769e-3
    y2 = y * y
    c = -1.1470745597729725e-11
    c = c * y2 + 2.08767569878681e-9
    c = c * y2 + -2.7557319223985893e-7
    c = c * y2 + 2.48015873015873e-5
    c = c * y2 + -1.3888888888888889e-3
    c = c * y2 + 4.1666666666666666e-2
    c = c * y2 + -0.5
    return c * y2 + 1.0


def _attn_body(eg, ef, ts, et, nbrs,
               tw, tb, Wq, Wk, Wv, Wo, Wf1, bf1, Wf2, bf2, out):
    f32 = jnp.float32
    dot = functools.partial(jnp.dot, preferred_element_type=f32)
    NBK = BB * K

    eg_ = eg[...]
    nb_emb = eg_[:NBK]
    cur = eg_[NBK:]

    tw_ = tw[...]
    tb_ = tb[...]
    src_te = _fast_cos(ts[...] * tw_ + tb_)
    nb_te = _fast_cos(et[...] * tw_ + tb_)

    Wq_ = Wq[...]
    q = dot(cur, Wq_[:D_EMB]) + dot(src_te, Wq_[D_EMB:])
    ef_ = ef[...][:NBK]
    Wk_ = Wk[...]
    kk = (dot(nb_emb, Wk_[:D_EMB]) + dot(ef_, Wk_[D_EMB:D_EMB + D_EDGE])
          + dot(nb_te, Wk_[D_EMB + D_EDGE:]))
    Wv_ = Wv[...]
    vv = (dot(nb_emb, Wv_[:D_EMB]) + dot(ef_, Wv_[D_EMB:D_EMB + D_EDGE])
          + dot(nb_te, Wv_[D_EMB + D_EDGE:]))

    pad = nbrs[...] == 0
    invalid = jnp.all(pad, axis=1, keepdims=True)
    kpos = lax.broadcasted_iota(jnp.int32, (BB, K), 1)
    mask = pad & jnp.logical_not(invalid & (kpos == 0))
    scale = 1.0 / (float(DH) ** 0.5)

    heads = []
    for h in range(N_HEADS):
        qh = q[:, h * DH:(h + 1) * DH]
        kh = kk[:, h * DH:(h + 1) * DH].reshape(BB, K, DH)
        vh = vv[:, h * DH:(h + 1) * DH].reshape(BB, K, DH)
        s = jnp.sum(kh * qh[:, None, :], axis=-1) * scale
        s = jnp.where(mask, -1e10, s)
        m = jnp.max(s, axis=-1, keepdims=True)
        e = jnp.exp(s - m)
        p = e / jnp.sum(e, axis=-1, keepdims=True)
        heads.append(jnp.sum(vh * p[:, :, None], axis=1))

    Wo_ = Wo[...]
    att = dot(heads[0], Wo_[:DH]) + dot(heads[1], Wo_[DH:])
    att = jnp.where(invalid, 0.0, att)

    Wf1_ = Wf1[...]
    h1 = jnp.maximum(dot(att, Wf1_[:QD]) + dot(cur, Wf1_[QD:]) + bf1[...], 0.0)
    out[...] = dot(h1, Wf2[...]) + bf2[...]


def _tc_compute(eg_g, ef_g, ts2, et2, nbrs,
                tw, tb, Wq, Wk, Wv, Wo, Wf1, bf1, Wf2, bf2):
    def im_blk(i):
        return (i, 0)

    def im_w(i):
        return (0, 0)

    def full(a):
        return pl.BlockSpec(a.shape, im_w)

    in_specs = [
        pl.BlockSpec((PB, D_EMB), im_blk),
        pl.BlockSpec((PB, D_EDGE), im_blk),
        pl.BlockSpec((BB, 1), im_blk),
        pl.BlockSpec((BB * K, 1), im_blk),
        pl.BlockSpec((BB, K), im_blk),
        full(tw), full(tb),
        full(Wq), full(Wk), full(Wv), full(Wo),
        full(Wf1), full(bf1), full(Wf2), full(bf2),
    ]
    return pl.pallas_call(
        _attn_body,
        grid=(NB,),
        in_specs=in_specs,
        out_specs=pl.BlockSpec((BB, D_EMB), im_blk),
        out_shape=jax.ShapeDtypeStruct((B, D_EMB), jnp.float32),
        compiler_params=pltpu.CompilerParams(
            dimension_semantics=("parallel",)),
    )(eg_g, ef_g, ts2, et2, nbrs,
      tw, tb, Wq, Wk, Wv, Wo, Wf1, bf1, Wf2, bf2)


def kernel(memory, source_nodes, timestamps, neighbors, edge_idxs, edge_times,
           node_features, edge_features, W_proj, b_proj, time_w, time_b,
           Wq, Wk, Wv, Wo, W_fc1, b_fc1, W_fc2, b_fc2):
    nbrs = neighbors.astype(jnp.int32)
    nidx = jnp.concatenate(
        [nbrs.reshape(NB, BB * K),
         source_nodes.astype(jnp.int32).reshape(NB, BB)],
        axis=1).reshape(1, NIDX)
    eidx = jnp.concatenate(
        [edge_idxs.astype(jnp.int32).reshape(NB, BB * K),
         jnp.zeros((NB, BB), jnp.int32)],
        axis=1).reshape(NIDX)

    ef_g = jnp.take(edge_features, eidx, axis=0)

    emb = _project_tables(node_features, memory,
                          W_proj, b_proj.reshape(1, D_EMB))
    eg_g = _sc_gather_emb(emb, nidx)

    return _tc_compute(
        eg_g, ef_g,
        timestamps.reshape(B, 1), edge_times.reshape(B * K, 1), nbrs,
        time_w.reshape(1, D_TIME), time_b.reshape(1, D_TIME),
        Wq, Wk, Wv, Wo,
        W_fc1, b_fc1.reshape(1, D_EMB), W_fc2, b_fc2.reshape(1, D_EMB))


# speedup vs baseline: 1.1288x; 1.1288x over previous
import functools

import jax
import jax.numpy as jnp
from jax import lax
from jax.experimental import pallas as pl
from jax.experimental.pallas import tpu as pltpu
from jax.experimental.pallas import tpu_sc as plsc

N_NODES = 100000
N_EDGES = 3200000
D_FEAT = 128
D_MEM = 128
D_EDGE = 16
D_TIME = 128
D_EMB = 128
N_HEADS = 2
B = 4096
K = 20
QD = D_EMB + D_TIME
KD = D_EMB + D_EDGE + D_TIME
DH = QD // N_HEADS

BB = 256
NB = B // BB
PB = BB * K + BB
NIDX = NB * PB
WIN = 128
N_CHUNKS = 1


def _sc_gather_nodes(node_features, memory_tbl, nidx, n_rows):
    mesh = plsc.VectorSubcoreMesh(core_axis_name="c", subcore_axis_name="s")

    @functools.partial(
        pl.kernel,
        out_type=(
            jax.ShapeDtypeStruct((n_rows, D_FEAT), jnp.float32),
            jax.ShapeDtypeStruct((n_rows, D_MEM), jnp.float32),
        ),
        mesh=mesh,
        scratch_types=[pltpu.SemaphoreType.DMA, pltpu.SemaphoreType.DMA],
    )
    def gather_kernel(nf_hbm, mem_hbm, nidx_hbm, nf_out, mem_out,
                      sem_a, sem_b):
        def body(nidx_v, nf_o, mem_o):
            ca = pltpu.make_async_copy(nf_hbm.at[nidx_v.at[0]], nf_o, sem_a)
            cb = pltpu.make_async_copy(mem_hbm.at[nidx_v.at[0]], mem_o, sem_b)
            ca.start()
            cb.start()
            ca.wait()
            cb.wait()

        pltpu.emit_pipeline(
            body,
            grid=(n_rows // WIN,),
            in_specs=[pl.BlockSpec((1, WIN), lambda i: (0, i))],
            out_specs=[
                pl.BlockSpec((WIN, D_FEAT), lambda i: (i, 0)),
                pl.BlockSpec((WIN, D_MEM), lambda i: (i, 0)),
            ],
            core_axis_name=("c", "s"),
            dimension_semantics=(pltpu.PARALLEL,),
        )(nidx_hbm, nf_out, mem_out)

    return gather_kernel(node_features, memory_tbl, nidx)


def _fast_cos(x):
    n = jnp.round(x * 0.15915494309189535)
    y = x - n * 6.28125 - n * 1.9353071795864769e-3
    y2 = y * y
    c = -1.1470745597729725e-11
    c = c * y2 + 2.08767569878681e-9
    c = c * y2 + -2.7557319223985893e-7
    c = c * y2 + 2.48015873015873e-5
    c = c * y2 + -1.3888888888888889e-3
    c = c * y2 + 4.1666666666666666e-2
    c = c * y2 + -0.5
    return c * y2 + 1.0


def _attn_body(nf, mem, ef, ts, et, nbrs,
               Wp, bp, tw, tb, Wq, Wk, Wv, Wo, Wf1, bf1, Wf2, bf2, out):
    f32 = jnp.float32
    bf16 = jnp.bfloat16
    dot = functools.partial(jnp.dot, preferred_element_type=f32)

    def dot16(a, b):
        return jnp.dot(a.astype(bf16), b.astype(bf16),
                       preferred_element_type=f32)

    NBK = BB * K

    Wp_ = Wp[...]
    bp_ = bp[...]
    nf_ = nf[...]
    mem_ = mem[...]
    nb_emb = (dot16(nf_[:NBK], Wp_[:D_FEAT])
              + dot16(mem_[:NBK], Wp_[D_FEAT:])
              + bp_)
    cur = (dot(nf_[NBK:], Wp_[:D_FEAT]) + dot(mem_[NBK:], Wp_[D_FEAT:])
           + bp_)

    tw_ = tw[...]
    tb_ = tb[...]
    src_te = _fast_cos(ts[...] * tw_ + tb_)
    nb_te = _fast_cos(et[...] * tw_ + tb_)

    Wq_ = Wq[...]
    q = dot(cur, Wq_[:D_EMB]) + dot(src_te, Wq_[D_EMB:])
    ef_ = ef[...][:NBK]
    Wk_ = Wk[...]
    kk = (dot16(nb_emb, Wk_[:D_EMB]) + dot16(ef_, Wk_[D_EMB:D_EMB + D_EDGE])
          + dot16(nb_te, Wk_[D_EMB + D_EDGE:]))
    Wv_ = Wv[...]
    vv = (dot16(nb_emb, Wv_[:D_EMB]) + dot16(ef_, Wv_[D_EMB:D_EMB + D_EDGE])
          + dot16(nb_te, Wv_[D_EMB + D_EDGE:]))

    pad = nbrs[...] == 0
    invalid = jnp.all(pad, axis=1, keepdims=True)
    kpos = lax.broadcasted_iota(jnp.int32, (BB, K), 1)
    mask = pad & jnp.logical_not(invalid & (kpos == 0))
    scale = 1.0 / (float(DH) ** 0.5)

    heads = []
    for h in range(N_HEADS):
        qh = q[:, h * DH:(h + 1) * DH]
        kh = kk[:, h * DH:(h + 1) * DH].reshape(BB, K, DH)
        vh = vv[:, h * DH:(h + 1) * DH].reshape(BB, K, DH)
        s = jnp.sum(kh * qh[:, None, :], axis=-1) * scale
        s = jnp.where(mask, -1e10, s)
        m = jnp.max(s, axis=-1, keepdims=True)
        e = jnp.exp(s - m)
        p = e / jnp.sum(e, axis=-1, keepdims=True)
        heads.append(jnp.sum(vh * p[:, :, None], axis=1))

    Wo_ = Wo[...]
    att = dot(heads[0], Wo_[:DH]) + dot(heads[1], Wo_[DH:])
    att = jnp.where(invalid, 0.0, att)

    Wf1_ = Wf1[...]
    h1 = jnp.maximum(dot(att, Wf1_[:QD]) + dot(cur, Wf1_[QD:]) + bf1[...], 0.0)
    out[...] = dot(h1, Wf2[...]) + bf2[...]


def _tc_compute(nf_g, mem_g, ef_g, ts2, et2, nbrs,
                Wp, bp, tw, tb, Wq, Wk, Wv, Wo, Wf1, bf1, Wf2, bf2,
                n_blocks):
    def im_blk(i):
        return (i, 0)

    def im_w(i):
        return (0, 0)

    def full(a):
        return pl.BlockSpec(a.shape, im_w)

    in_specs = [
        pl.BlockSpec((PB, D_FEAT), im_blk),
        pl.BlockSpec((PB, D_MEM), im_blk),
        pl.BlockSpec((PB, D_EDGE), im_blk),
        pl.BlockSpec((BB, 1), im_blk),
        pl.BlockSpec((BB * K, 1), im_blk),
        pl.BlockSpec((BB, K), im_blk),
        full(Wp), full(bp), full(tw), full(tb),
        full(Wq), full(Wk), full(Wv), full(Wo),
        full(Wf1), full(bf1), full(Wf2), full(bf2),
    ]
    return pl.pallas_call(
        _attn_body,
        grid=(n_blocks,),
        in_specs=in_specs,
        out_specs=pl.BlockSpec((BB, D_EMB), im_blk),
        out_shape=jax.ShapeDtypeStruct((n_blocks * BB, D_EMB), jnp.float32),
        compiler_params=pltpu.CompilerParams(
            dimension_semantics=("parallel",)),
    )(nf_g, mem_g, ef_g, ts2, et2, nbrs,
      Wp, bp, tw, tb, Wq, Wk, Wv, Wo, Wf1, bf1, Wf2, bf2)


def kernel(memory, source_nodes, timestamps, neighbors, edge_idxs, edge_times,
           node_features, edge_features, W_proj, b_proj, time_w, time_b,
           Wq, Wk, Wv, Wo, W_fc1, b_fc1, W_fc2, b_fc2):
    nbrs = neighbors.astype(jnp.int32)
    nidx = jnp.concatenate(
        [nbrs.reshape(NB, BB * K),
         source_nodes.astype(jnp.int32).reshape(NB, BB)],
        axis=1)
    eidx = jnp.concatenate(
        [edge_idxs.astype(jnp.int32).reshape(NB, BB * K),
         jnp.zeros((NB, BB), jnp.int32)],
        axis=1)

    ts2 = timestamps.reshape(B, 1)
    et2 = edge_times.reshape(B * K, 1)
    weights = (W_proj, b_proj.reshape(1, D_EMB),
               time_w.reshape(1, D_TIME), time_b.reshape(1, D_TIME),
               Wq, Wk, Wv, Wo,
               W_fc1, b_fc1.reshape(1, D_EMB), W_fc2, b_fc2.reshape(1, D_EMB))

    blk = NB // N_CHUNKS
    outs = []
    for h in range(N_CHUNKS):
        nidx_h = nidx[h * blk:(h + 1) * blk].reshape(1, blk * PB)
        eidx_h = eidx[h * blk:(h + 1) * blk].reshape(blk * PB)
        nf_g, mem_g = _sc_gather_nodes(node_features, memory, nidx_h,
                                       blk * PB)
        ef_g = jnp.take(edge_features, eidx_h, axis=0)
        outs.append(_tc_compute(
            nf_g, mem_g, ef_g,
            ts2[h * blk * BB:(h + 1) * blk * BB],
            et2[h * blk * BB * K:(h + 1) * blk * BB * K],
            nbrs[h * blk * BB:(h + 1) * blk * BB],
            *weights, n_blocks=blk))
    return jnp.concatenate(outs, axis=0)


# speedup vs baseline: 1.1352x; 1.0057x over previous
import functools

import jax
import jax.numpy as jnp
from jax import lax
from jax.experimental import pallas as pl
from jax.experimental.pallas import tpu as pltpu
from jax.experimental.pallas import tpu_sc as plsc

N_NODES = 100000
N_EDGES = 3200000
D_FEAT = 128
D_MEM = 128
D_EDGE = 16
D_TIME = 128
D_EMB = 128
N_HEADS = 2
B = 4096
K = 20
QD = D_EMB + D_TIME
KD = D_EMB + D_EDGE + D_TIME
DH = QD // N_HEADS

BB = 256
NB = B // BB
PB = BB * K + BB
NIDX = NB * PB
WIN = 128
N_CHUNKS = 1


def _sc_gather_nodes(node_features, memory_tbl, nidx, n_rows):
    mesh = plsc.VectorSubcoreMesh(core_axis_name="c", subcore_axis_name="s")

    @functools.partial(
        pl.kernel,
        out_type=(
            jax.ShapeDtypeStruct((n_rows, D_FEAT), jnp.float32),
            jax.ShapeDtypeStruct((n_rows, D_MEM), jnp.float32),
        ),
        mesh=mesh,
        scratch_types=[pltpu.SemaphoreType.DMA, pltpu.SemaphoreType.DMA],
    )
    def gather_kernel(nf_hbm, mem_hbm, nidx_hbm, nf_out, mem_out,
                      sem_a, sem_b):
        def body(nidx_v, nf_o, mem_o):
            ca = pltpu.make_async_copy(nf_hbm.at[nidx_v.at[0]], nf_o, sem_a)
            cb = pltpu.make_async_copy(mem_hbm.at[nidx_v.at[0]], mem_o, sem_b)
            ca.start()
            cb.start()
            ca.wait()
            cb.wait()

        pltpu.emit_pipeline(
            body,
            grid=(n_rows // WIN,),
            in_specs=[pl.BlockSpec((1, WIN), lambda i: (0, i))],
            out_specs=[
                pl.BlockSpec((WIN, D_FEAT), lambda i: (i, 0)),
                pl.BlockSpec((WIN, D_MEM), lambda i: (i, 0)),
            ],
            core_axis_name=("c", "s"),
            dimension_semantics=(pltpu.PARALLEL,),
        )(nidx_hbm, nf_out, mem_out)

    return gather_kernel(node_features, memory_tbl, nidx)


def _fast_cos(x):
    n = jnp.round(x * 0.15915494309189535)
    y = x - n * 6.28125 - n * 1.9353071795864769e-3
    y2 = y * y
    c = -1.1470745597729725e-11
    c = c * y2 + 2.08767569878681e-9
    c = c * y2 + -2.7557319223985893e-7
    c = c * y2 + 2.48015873015873e-5
    c = c * y2 + -1.3888888888888889e-3
    c = c * y2 + 4.1666666666666666e-2
    c = c * y2 + -0.5
    return c * y2 + 1.0


def _attn_body(nf, mem, ef, ts, et, nbrs,
               Wp, bp, tw, tb, Wq, Wk, Wv, Wo, Wf1, bf1, Wf2, bf2, out):
    f32 = jnp.float32
    dot = functools.partial(jnp.dot, preferred_element_type=f32)
    NBK = BB * K

    Wp_ = Wp[...]
    bp_ = bp[...]
    nf_ = nf[...]
    mem_ = mem[...]
    nb_emb = (dot(nf_[:NBK], Wp_[:D_FEAT]) + dot(mem_[:NBK], Wp_[D_FEAT:])
              + bp_)
    cur = (dot(nf_[NBK:], Wp_[:D_FEAT]) + dot(mem_[NBK:], Wp_[D_FEAT:])
           + bp_)

    tw_ = tw[...]
    tb_ = tb[...]
    src_te = _fast_cos(ts[...] * tw_ + tb_)
    nb_te = _fast_cos(et[...] * tw_ + tb_)

    Wq_ = Wq[...]
    q = dot(cur, Wq_[:D_EMB]) + dot(src_te, Wq_[D_EMB:])
    ef_ = ef[...][:NBK]
    Wk_ = Wk[...]
    kk = (dot(nb_emb, Wk_[:D_EMB]) + dot(ef_, Wk_[D_EMB:D_EMB + D_EDGE])
          + dot(nb_te, Wk_[D_EMB + D_EDGE:]))
    Wv_ = Wv[...]
    vv = (dot(nb_emb, Wv_[:D_EMB]) + dot(ef_, Wv_[D_EMB:D_EMB + D_EDGE])
          + dot(nb_te, Wv_[D_EMB + D_EDGE:]))

    pad = nbrs[...] == 0
    invalid = jnp.all(pad, axis=1, keepdims=True)
    kpos = lax.broadcasted_iota(jnp.int32, (BB, K), 1)
    mask = pad & jnp.logical_not(invalid & (kpos == 0))
    scale = 1.0 / (float(DH) ** 0.5)

    heads = []
    for h in range(N_HEADS):
        qh = q[:, h * DH:(h + 1) * DH]
        kh = kk[:, h * DH:(h + 1) * DH].reshape(BB, K, DH)
        vh = vv[:, h * DH:(h + 1) * DH].reshape(BB, K, DH)
        s = jnp.sum(kh * qh[:, None, :], axis=-1) * scale
        s = jnp.where(mask, -1e10, s)
        m = jnp.max(s, axis=-1, keepdims=True)
        e = jnp.exp(s - m)
        p = e / jnp.sum(e, axis=-1, keepdims=True)
        heads.append(jnp.sum(vh * p[:, :, None], axis=1))

    Wo_ = Wo[...]
    att = dot(heads[0], Wo_[:DH]) + dot(heads[1], Wo_[DH:])
    att = jnp.where(invalid, 0.0, att)

    Wf1_ = Wf1[...]
    h1 = jnp.maximum(dot(att, Wf1_[:QD]) + dot(cur, Wf1_[QD:]) + bf1[...], 0.0)
    out[...] = dot(h1, Wf2[...]) + bf2[...]


def _tc_compute(nf_g, mem_g, ef_g, ts2, et2, nbrs,
                Wp, bp, tw, tb, Wq, Wk, Wv, Wo, Wf1, bf1, Wf2, bf2,
                n_blocks):
    def im_blk(i):
        return (i, 0)

    def im_w(i):
        return (0, 0)

    def full(a):
        return pl.BlockSpec(a.shape, im_w)

    in_specs = [
        pl.BlockSpec((PB, D_FEAT), im_blk),
        pl.BlockSpec((PB, D_MEM), im_blk),
        pl.BlockSpec((PB, D_EDGE), im_blk),
        pl.BlockSpec((BB, 1), im_blk),
        pl.BlockSpec((BB * K, 1), im_blk),
        pl.BlockSpec((BB, K), im_blk),
        full(Wp), full(bp), full(tw), full(tb),
        full(Wq), full(Wk), full(Wv), full(Wo),
        full(Wf1), full(bf1), full(Wf2), full(bf2),
    ]
    return pl.pallas_call(
        _attn_body,
        grid=(n_blocks,),
        in_specs=in_specs,
        out_specs=pl.BlockSpec((BB, D_EMB), im_blk),
        out_shape=jax.ShapeDtypeStruct((n_blocks * BB, D_EMB), jnp.float32),
        compiler_params=pltpu.CompilerParams(
            dimension_semantics=("parallel",)),
    )(nf_g, mem_g, ef_g, ts2, et2, nbrs,
      Wp, bp, tw, tb, Wq, Wk, Wv, Wo, Wf1, bf1, Wf2, bf2)


def kernel(memory, source_nodes, timestamps, neighbors, edge_idxs, edge_times,
           node_features, edge_features, W_proj, b_proj, time_w, time_b,
           Wq, Wk, Wv, Wo, W_fc1, b_fc1, W_fc2, b_fc2):
    nbrs = neighbors.astype(jnp.int32)
    nidx = jnp.concatenate(
        [nbrs.reshape(NB, BB * K),
         source_nodes.astype(jnp.int32).reshape(NB, BB)],
        axis=1)
    eidx = jnp.concatenate(
        [edge_idxs.astype(jnp.int32).reshape(NB, BB * K),
         jnp.zeros((NB, BB), jnp.int32)],
        axis=1)

    ts2 = timestamps.reshape(B, 1)
    et2 = edge_times.reshape(B * K, 1)
    weights = (W_proj, b_proj.reshape(1, D_EMB),
               time_w.reshape(1, D_TIME), time_b.reshape(1, D_TIME),
               Wq, Wk, Wv, Wo,
               W_fc1, b_fc1.reshape(1, D_EMB), W_fc2, b_fc2.reshape(1, D_EMB))

    blk = NB // N_CHUNKS
    outs = []
    for h in range(N_CHUNKS):
        nidx_h = nidx[h * blk:(h + 1) * blk].reshape(1, blk * PB)
        eidx_h = eidx[h * blk:(h + 1) * blk].reshape(blk * PB)
        nf_g, mem_g = _sc_gather_nodes(node_features, memory, nidx_h,
                                       blk * PB)
        ef_g = jnp.take(edge_features, eidx_h, axis=0)
        outs.append(_tc_compute(
            nf_g, mem_g, ef_g,
            ts2[h * blk * BB:(h + 1) * blk * BB],
            et2[h * blk * BB * K:(h + 1) * blk * BB * K],
            nbrs[h * blk * BB:(h + 1) * blk * BB],
            *weights, n_blocks=blk))
    return jnp.concatenate(outs, axis=0)
